# baseline (speedup 1.0000x reference)
import functools

import jax
import jax.numpy as jnp
from jax.experimental import pallas as pl
from jax.experimental.pallas import tpu as pltpu

D_MODEL_ = 4096
N_EXPERTS_ = 64
K_ = 8
BT_ = 1024


def _router_block(x_ref, w_ref, ew_ref, ei_ref, hist_ref, sc_ref):
    i = pl.program_id(0)

    scores = sc_ref[(i + 1) % 2]
    lane = jax.lax.broadcasted_iota(jnp.int32, scores.shape, 1)
    lane_rev = (N_EXPERTS_ - 1 - lane).astype(jnp.float32)
    neg_one = jnp.float32(-1.0)

    ws = []
    idxs = []
    cur = scores
    for _ in range(K_):
        mx = jnp.max(cur, axis=-1, keepdims=True)
        rev = jnp.max(jnp.where(cur == mx, lane_rev, neg_one),
                      axis=-1, keepdims=True)
        idx = (N_EXPERTS_ - 1) - rev.astype(jnp.int32)
        pick = lane == idx
        cur = jnp.where(pick, neg_one, cur)
        ws.append(mx)
        idxs.append(idx)

    ew_ref[...] = jnp.concatenate(ws, axis=-1)
    ei_ref[...] = jnp.concatenate(idxs, axis=-1)

    contrib = jnp.sum((cur < 0).astype(jnp.int32), axis=0, keepdims=True)

    @pl.when(i == 1)
    def _init():
        hist_ref[...] = contrib

    @pl.when(i > 1)
    def _accum():
        hist_ref[...] += contrib

    logits = jnp.dot(x_ref[...], w_ref[...],
                     preferred_element_type=jnp.float32)
    m = jnp.max(logits, axis=-1, keepdims=True)
    e = jnp.exp(logits - m)
    inv = 1.0 / jnp.sum(e, axis=-1, keepdims=True)
    sc_ref[i % 2] = e * inv


@functools.partial(jax.jit, static_argnames=())
def kernel(x, W):
    n_tokens = x.shape[0]
    nb = n_tokens // BT_
    grid = (nb + 1,)
    last = nb - 1
    ew, ei, hist = pl.pallas_call(
        _router_block,
        grid=grid,
        in_specs=[
            pl.BlockSpec((BT_, D_MODEL_),
                         lambda i: (jnp.minimum(i, last), 0)),
            pl.BlockSpec((D_MODEL_, N_EXPERTS_), lambda i: (0, 0)),
        ],
        out_specs=[
            pl.BlockSpec((BT_, K_), lambda i: (jnp.maximum(i - 1, 0), 0)),
            pl.BlockSpec((BT_, K_), lambda i: (jnp.maximum(i - 1, 0), 0)),
            pl.BlockSpec((1, N_EXPERTS_), lambda i: (0, 0)),
        ],
        out_shape=[
            jax.ShapeDtypeStruct((n_tokens, K_), jnp.float32),
            jax.ShapeDtypeStruct((n_tokens, K_), jnp.int32),
            jax.ShapeDtypeStruct((1, N_EXPERTS_), jnp.int32),
        ],
        scratch_shapes=[pltpu.VMEM((2, BT_, N_EXPERTS_), jnp.float32)],
        compiler_params=pltpu.CompilerParams(
            dimension_semantics=("arbitrary",),
        ),
    )(x, W)
    return ew, ei, hist.reshape(N_EXPERTS_)


# speedup vs baseline: 1.1146x; 1.1146x over previous
import functools

import jax
import jax.numpy as jnp
from jax.experimental import pallas as pl
from jax.experimental.pallas import tpu as pltpu

D_MODEL_ = 4096
N_EXPERTS_ = 64
K_ = 8
BT_ = 1024


def _router_block(x_ref, w_ref, ew_ref, ei_ref, hist_ref):
    logitsT = jax.lax.dot_general(
        w_ref[...], x_ref[...],
        dimension_numbers=(((0,), (1,)), ((), ())),
        preferred_element_type=jnp.float32)
    m = jnp.max(logitsT, axis=0, keepdims=True)
    e = jnp.exp(logitsT - m)
    scores = e / jnp.sum(e, axis=0, keepdims=True)

    sub = jax.lax.broadcasted_iota(jnp.int32, scores.shape, 0)
    sub_rev = (N_EXPERTS_ - 1 - sub).astype(jnp.float32)
    neg_one = jnp.float32(-1.0)

    ws = []
    idxs = []
    cur = scores
    for _ in range(K_):
        mx = jnp.max(cur, axis=0, keepdims=True)
        rev = jnp.max(jnp.where(cur == mx, sub_rev, neg_one),
                      axis=0, keepdims=True)
        idx = (N_EXPERTS_ - 1) - rev.astype(jnp.int32)
        pick = sub == idx
        cur = jnp.where(pick, neg_one, cur)
        ws.append(mx)
        idxs.append(idx)

    ew_ref[...] = jnp.concatenate(ws, axis=0).T
    ei_ref[...] = jnp.concatenate(idxs, axis=0).T

    contrib = jnp.sum((cur < 0).astype(jnp.int32), axis=1, keepdims=True).T

    @pl.when(pl.program_id(0) == 0)
    def _init():
        hist_ref[...] = jnp.zeros_like(hist_ref)

    hist_ref[...] += contrib


@functools.partial(jax.jit, static_argnames=())
def kernel(x, W):
    n_tokens = x.shape[0]
    grid = (n_tokens // BT_,)
    ew, ei, hist = pl.pallas_call(
        _router_block,
        grid=grid,
        in_specs=[
            pl.BlockSpec((BT_, D_MODEL_), lambda i: (i, 0)),
            pl.BlockSpec((D_MODEL_, N_EXPERTS_), lambda i: (0, 0)),
        ],
        out_specs=[
            pl.BlockSpec((BT_, K_), lambda i: (i, 0)),
            pl.BlockSpec((BT_, K_), lambda i: (i, 0)),
            pl.BlockSpec((1, N_EXPERTS_), lambda i: (0, 0)),
        ],
        out_shape=[
            jax.ShapeDtypeStruct((n_tokens, K_), jnp.float32),
            jax.ShapeDtypeStruct((n_tokens, K_), jnp.int32),
            jax.ShapeDtypeStruct((1, N_EXPERTS_), jnp.int32),
        ],
        compiler_params=pltpu.CompilerParams(
            dimension_semantics=("arbitrary",),
        ),
    )(x, W)
    return ew, ei, hist.reshape(N_EXPERTS_)
